# Initial kernel scaffold; baseline (speedup 1.0000x reference)
import jax
import jax.numpy as jnp
from jax.experimental import pallas as pl


def kernel(x, B, C, A):
    raise NotImplementedError("write your pallas kernel here")


# speedup vs baseline: 32.8702x; 32.8702x over previous
import jax
import jax.numpy as jnp
from jax.experimental import pallas as pl
from jax.experimental.pallas import tpu as pltpu

_SEQ_BLK = 512
_UNROLL = 8


def _scan_body(b_ref, c_ref, a_ref, y_ref, h_ref):
    s = pl.program_id(1)

    @pl.when(s == 0)
    def _():
        h_ref[...] = jnp.zeros_like(h_ref)

    def body(i, h):
        t0 = i * _UNROLL
        for u in range(_UNROLL):
            t = t0 + u
            h = a_ref[:, t] * h + b_ref[:, t]
            y_ref[:, t] = c_ref[:, t] * h
        return h

    h = jax.lax.fori_loop(0, _SEQ_BLK // _UNROLL, body, h_ref[...])
    h_ref[...] = h


@jax.jit
def kernel(x, B, C, A):
    del x
    batch, seq_len, state_dim = B.shape
    sub = 8
    lanes = state_dim // sub
    bb = 2
    B4 = B.reshape(batch, seq_len, sub, lanes)
    C4 = C.reshape(batch, seq_len, sub, lanes)
    A4 = A.reshape(batch, seq_len, sub, lanes)

    blk = (bb, _SEQ_BLK, sub, lanes)
    spec = pl.BlockSpec(blk, lambda p, s: (p, s, 0, 0))

    y = pl.pallas_call(
        _scan_body,
        grid=(batch // bb, seq_len // _SEQ_BLK),
        in_specs=[spec, spec, spec],
        out_specs=spec,
        out_shape=jax.ShapeDtypeStruct((batch, seq_len, sub, lanes), B.dtype),
        scratch_shapes=[pltpu.VMEM((bb, sub, lanes), jnp.float32)],
        compiler_params=pltpu.CompilerParams(
            dimension_semantics=("parallel", "arbitrary"),
        ),
    )(B4, C4, A4)
    return y.reshape(batch, seq_len, state_dim)


# speedup vs baseline: 116.4939x; 3.5441x over previous
import jax
import jax.numpy as jnp
from jax.experimental import pallas as pl
from jax.experimental.pallas import tpu as pltpu

_SEQ_BLK = 512
_SUB = 8


def _scan_body(b_ref, c_ref, a_ref, y_ref, h_ref):
    s = pl.program_id(1)

    @pl.when(s == 0)
    def _():
        h_ref[...] = jnp.zeros_like(h_ref)

    iota = jax.lax.broadcasted_iota(jnp.int32, (1, _SUB, 1), 1)

    def chunk(c, h):
        r = pl.ds(c * _SUB, _SUB)
        A = a_ref[:, r, :]
        Bv = b_ref[:, r, :]
        for k in (1, 2, 4):
            mask = iota < k
            A_sh = jnp.where(mask, 1.0, jnp.roll(A, k, axis=1))
            B_sh = jnp.where(mask, 0.0, jnp.roll(Bv, k, axis=1))
            Bv = A * B_sh + Bv
            A = A * A_sh
        hr = A * h + Bv
        y_ref[:, r, :] = c_ref[:, r, :] * hr
        return hr[:, _SUB - 1 : _SUB, :]

    h = jax.lax.fori_loop(0, _SEQ_BLK // _SUB, chunk, h_ref[...])
    h_ref[...] = h


@jax.jit
def kernel(x, B, C, A):
    del x
    batch, seq_len, state_dim = B.shape
    bb = 2

    blk = (bb, _SEQ_BLK, state_dim)
    spec = pl.BlockSpec(blk, lambda p, s: (p, s, 0))

    return pl.pallas_call(
        _scan_body,
        grid=(batch // bb, seq_len // _SEQ_BLK),
        in_specs=[spec, spec, spec],
        out_specs=spec,
        out_shape=jax.ShapeDtypeStruct((batch, seq_len, state_dim), B.dtype),
        scratch_shapes=[pltpu.VMEM((bb, 1, state_dim), jnp.float32)],
        compiler_params=pltpu.CompilerParams(
            dimension_semantics=("parallel", "arbitrary"),
        ),
    )(B, C, A)
